# baseline (speedup 1.0000x reference)
import functools

import jax
import jax.numpy as jnp
from jax import lax
from jax.experimental import pallas as pl
from jax.experimental.pallas import tpu as pltpu
from jax.experimental.pallas import tpu_sc as plsc

B, A, NBH, DIN, F, NB = 8, 1250, 32, 128, 128, 50
CUTOFF = 1.0
ROWS = B * A * NBH

_NC, _NS = 2, 16
_NW = _NC * _NS
_BPW = ROWS // _NW
_CH = 80
_NCHUNK = _BPW // _CH
_WPB = _NW // B

_BLK = 125
_ABLK = A // _BLK


def _sc_gather(table, idx):
    mesh = plsc.VectorSubcoreMesh(core_axis_name="c", subcore_axis_name="s")

    @functools.partial(
        pl.kernel,
        mesh=mesh,
        out_type=jax.ShapeDtypeStruct((ROWS, DIN), table.dtype),
        scratch_types=[
            pltpu.VMEM((_BPW,), jnp.int32),
            pltpu.VMEM((_CH, DIN), table.dtype),
            pltpu.VMEM((_CH, DIN), table.dtype),
            pltpu.SemaphoreType.DMA,
            pltpu.SemaphoreType.DMA,
            pltpu.SemaphoreType.DMA,
            pltpu.SemaphoreType.DMA,
        ],
    )
    def gather_kernel(table_hbm, idx_hbm, out_hbm, idx_v, rows0, rows1,
                      sg0, sg1, sw0, sw1):
        wid = lax.axis_index("s") * _NC + lax.axis_index("c")
        base = wid * _BPW
        boff = (wid // _WPB) * A

        pltpu.sync_copy(idx_hbm.at[pl.ds(base, _BPW)], idx_v)

        @pl.loop(0, _BPW // 16)
        def _(j):
            idx_v[pl.ds(j * 16, 16)] = idx_v[pl.ds(j * 16, 16)] + boff

        rows = (rows0, rows1)
        sg = (sg0, sg1)
        sw = (sw0, sw1)

        def start_gather(c, k):
            pltpu.async_copy(
                table_hbm.at[idx_v.at[pl.ds(c * _CH, _CH)]], rows[k], sg[k])

        def wait_gather(c, k):
            pltpu.make_async_copy(
                table_hbm.at[idx_v.at[pl.ds(c * _CH, _CH)]], rows[k],
                sg[k]).wait()

        def start_write(c, k):
            pltpu.async_copy(rows[k], out_hbm.at[pl.ds(base + c * _CH, _CH)],
                             sw[k])

        def wait_write(k):
            pltpu.make_async_copy(rows[k], out_hbm.at[pl.ds(base, _CH)],
                                  sw[k]).wait()

        start_gather(0, 0)

        @pl.loop(0, _NCHUNK, step=2)
        def _(c):
            for k in (0, 1):
                cc = c + k
                other = 1 - k

                @pl.when(cc < _NCHUNK)
                def _():
                    @pl.when(cc + 1 < _NCHUNK)
                    def _():
                        @pl.when(cc >= 1)
                        def _():
                            wait_write(other)

                        start_gather(cc + 1, other)

                    wait_gather(cc, k)
                    start_write(cc, k)

        wait_write(0)
        wait_write(1)

    return gather_kernel(table, idx)


def _filter_body(f_ref, r_ref, pm_ref, w1_ref, b1_ref, w2_ref, b2_ref,
                 wf_ref, mask_ref):
    f32 = jnp.float32
    ff = 2.0 * f_ref[0].reshape(_BLK * NBH, NB) - 1.0
    h = jnp.dot(ff, w1_ref[...], preferred_element_type=f32) + b1_ref[...]
    h = jnp.maximum(h, 0.0) + jnp.log1p(jnp.exp(-jnp.abs(h))) - jnp.log(2.0)
    wf = jnp.dot(h, w2_ref[...], preferred_element_type=f32) + b2_ref[...]
    r = r_ref[0]
    cut = 0.5 * (jnp.cos(jnp.pi * r / CUTOFF) + 1.0) * (r < CUTOFF).astype(f32)
    wf = wf.reshape(_BLK, NBH, F) * cut[:, :, None]
    wf_ref[...] = wf[None]
    mask_ref[...] = (pm_ref[0] * cut)[None]


def _attn_body(x_ref, e_ref, t_ref, mask_ref, wf_ref, xn_ref,
               wq_ref, wkt_ref, wv_ref, wo_ref, m_ref):
    f32 = jnp.float32
    x = x_ref[0]
    xc = x + e_ref[0] + t_ref[0]
    q = jnp.dot(xc, wq_ref[...], preferred_element_type=f32)
    p = jnp.dot(q, wkt_ref[...], preferred_element_type=f32)
    vs = jnp.dot(x, wv_ref[...], preferred_element_type=f32)
    xn = xn_ref[...]
    wf = wf_ref[0]
    vn = jnp.dot(xn, wv_ref[...], preferred_element_type=f32).reshape(_BLK, NBH, F)
    vn = vn * wf

    inv = 1.0 / jnp.sqrt(f32(F))
    sn = jnp.sum(p[:, None, :] * xn.reshape(_BLK, NBH, F), axis=-1) * inv
    ss = jnp.sum(p * x, axis=-1, keepdims=True) * inv
    sn = jnp.where(mask_ref[0] > 0.0, sn, -1e9)
    mx = jnp.maximum(jnp.max(sn, axis=-1, keepdims=True), ss)
    en = jnp.exp(sn - mx)
    es = jnp.exp(ss - mx)
    den = es + jnp.sum(en, axis=-1, keepdims=True)
    attn_n = en / den
    attn_s = es / den
    mpre = attn_s * vs + jnp.sum(attn_n[:, :, None] * vn, axis=1)
    m_ref[...] = jnp.dot(mpre, wo_ref[...], preferred_element_type=f32)[None]


_atom3 = lambda n: pl.BlockSpec((1, _BLK, n), lambda b, j: (b * _ABLK + j, 0, 0))
_full = lambda shape: pl.BlockSpec(shape, lambda b, j: tuple(0 for _ in shape))
_wf4 = pl.BlockSpec((1, _BLK, NBH, F), lambda b, j: (b, j, 0, 0))


def _filter_call(interpret=False):
    return pl.pallas_call(
        _filter_body,
        grid=(B, _ABLK),
        in_specs=[
            pl.BlockSpec((1, _BLK, NBH, NB), lambda b, j: (b, j, 0, 0)),
            _atom3(NBH),
            _atom3(NBH),
            _full((NB, F)),
            _full((F,)),
            _full((F, F)),
            _full((F,)),
        ],
        out_specs=[
            _wf4,
            _atom3(NBH),
        ],
        out_shape=[
            jax.ShapeDtypeStruct((B, A, NBH, F), jnp.float32),
            jax.ShapeDtypeStruct((B * _ABLK, _BLK, NBH), jnp.float32),
        ],
        interpret=interpret,
    )


def _attn_call(interpret=False):
    return pl.pallas_call(
        _attn_body,
        grid=(B, _ABLK),
        in_specs=[
            _atom3(DIN),
            _atom3(DIN),
            pl.BlockSpec((1, 1, DIN), lambda b, j: (b, 0, 0)),
            _atom3(NBH),
            _wf4,
            pl.BlockSpec((_BLK * NBH, DIN), lambda b, j: (b * _ABLK + j, 0)),
            _full((DIN, F)),
            _full((F, DIN)),
            _full((DIN, F)),
            _full((F, F)),
        ],
        out_specs=[_atom3(F)],
        out_shape=[jax.ShapeDtypeStruct((B * _ABLK, _BLK, F), jnp.float32)],
        interpret=interpret,
    )


def kernel(e, x, t, r_ij, neighbors, pairwise_mask, f_ij, W1, b1, W2, b2, Wq, Wk, Wv, Wo):
    x2d = x.reshape(B * A, DIN)
    idx = neighbors.reshape(ROWS)
    nb = B * _ABLK
    x3 = x.reshape(nb, _BLK, DIN)
    e3 = e.reshape(nb, _BLK, DIN)
    r3 = r_ij.reshape(nb, _BLK, NBH)
    pm3 = pairwise_mask.reshape(nb, _BLK, NBH)

    xn = _sc_gather(x2d, idx)
    wf, mask3 = _filter_call()(f_ij, r3, pm3, W1, b1, W2, b2)
    (m,) = _attn_call()(
        x3, e3, t.reshape(B, 1, DIN), mask3, wf, xn,
        Wq, Wk.T, Wv, Wo)
    return m.reshape(B, A, F), wf


# speedup vs baseline: 1.1543x; 1.1543x over previous
import functools

import jax
import jax.numpy as jnp
from jax import lax
from jax.experimental import pallas as pl
from jax.experimental.pallas import tpu as pltpu
from jax.experimental.pallas import tpu_sc as plsc

B, A, NBH, DIN, F, NB = 8, 1250, 32, 128, 128, 50
CUTOFF = 1.0
ROWS = B * A * NBH

_NC, _NS = 2, 16
_NW = _NC * _NS
_BPW = ROWS // _NW
_CH = 80
_NCHUNK = _BPW // _CH
_WPB = _NW // B

_BLK = 250
_ABLK = A // _BLK


def _sc_gather(table, idx):
    mesh = plsc.VectorSubcoreMesh(core_axis_name="c", subcore_axis_name="s")

    @functools.partial(
        pl.kernel,
        mesh=mesh,
        out_type=jax.ShapeDtypeStruct((ROWS, DIN), table.dtype),
        scratch_types=[
            pltpu.VMEM((_BPW,), jnp.int32),
            pltpu.VMEM((_CH, DIN), table.dtype),
            pltpu.VMEM((_CH, DIN), table.dtype),
            pltpu.SemaphoreType.DMA,
            pltpu.SemaphoreType.DMA,
            pltpu.SemaphoreType.DMA,
            pltpu.SemaphoreType.DMA,
        ],
    )
    def gather_kernel(table_hbm, idx_hbm, out_hbm, idx_v, rows0, rows1,
                      sg0, sg1, sw0, sw1):
        wid = lax.axis_index("s") * _NC + lax.axis_index("c")
        base = wid * _BPW
        boff = (wid // _WPB) * A

        pltpu.sync_copy(idx_hbm.at[pl.ds(base, _BPW)], idx_v)

        @pl.loop(0, _BPW // 16)
        def _(j):
            idx_v[pl.ds(j * 16, 16)] = idx_v[pl.ds(j * 16, 16)] + boff

        rows = (rows0, rows1)
        sg = (sg0, sg1)
        sw = (sw0, sw1)

        def start_gather(c, k):
            pltpu.async_copy(
                table_hbm.at[idx_v.at[pl.ds(c * _CH, _CH)]], rows[k], sg[k])

        def wait_gather(c, k):
            pltpu.make_async_copy(
                table_hbm.at[idx_v.at[pl.ds(c * _CH, _CH)]], rows[k],
                sg[k]).wait()

        def start_write(c, k):
            pltpu.async_copy(rows[k], out_hbm.at[pl.ds(base + c * _CH, _CH)],
                             sw[k])

        def wait_write(k):
            pltpu.make_async_copy(rows[k], out_hbm.at[pl.ds(base, _CH)],
                                  sw[k]).wait()

        start_gather(0, 0)

        @pl.loop(0, _NCHUNK, step=2)
        def _(c):
            for k in (0, 1):
                cc = c + k
                other = 1 - k

                @pl.when(cc < _NCHUNK)
                def _():
                    @pl.when(cc + 1 < _NCHUNK)
                    def _():
                        @pl.when(cc >= 1)
                        def _():
                            wait_write(other)

                        start_gather(cc + 1, other)

                    wait_gather(cc, k)
                    start_write(cc, k)

        wait_write(0)
        wait_write(1)

    return gather_kernel(table, idx)


def _tc_body(x_ref, e_ref, t_ref, r_ref, pm_ref, f_ref, xn_ref,
             w1_ref, b1_ref, w2_ref, b2_ref, wq_ref, wkt_ref, wv_ref, wo_ref,
             wf_ref, m_ref):
    f32 = jnp.float32
    ff = 2.0 * f_ref[0].reshape(_BLK * NBH, NB) - 1.0
    h = jnp.dot(ff, w1_ref[...], preferred_element_type=f32) + b1_ref[...]
    h = jnp.maximum(h, 0.0) + jnp.log1p(jnp.exp(-jnp.abs(h))) - jnp.log(2.0)
    wf = jnp.dot(h, w2_ref[...], preferred_element_type=f32) + b2_ref[...]
    r = r_ref[0]
    cut = 0.5 * (jnp.cos(jnp.pi * r / CUTOFF) + 1.0) * (r < CUTOFF).astype(f32)
    wf = wf.reshape(_BLK, NBH, F) * cut[:, :, None]
    wf_ref[...] = wf[None]

    x = x_ref[0]
    xc = x + e_ref[0] + t_ref[0]
    q = jnp.dot(xc, wq_ref[...], preferred_element_type=f32)
    p = jnp.dot(q, wkt_ref[...], preferred_element_type=f32)
    vs = jnp.dot(x, wv_ref[...], preferred_element_type=f32)
    xn = xn_ref[...]
    vn = jnp.dot(xn, wv_ref[...], preferred_element_type=f32).reshape(_BLK, NBH, F)
    vn = vn * wf

    inv = 1.0 / jnp.sqrt(f32(F))
    sn = jnp.sum(p[:, None, :] * xn.reshape(_BLK, NBH, F), axis=-1) * inv
    ss = jnp.sum(p * x, axis=-1, keepdims=True) * inv
    sn = jnp.where(pm_ref[0] * cut > 0.0, sn, -1e9)
    mx = jnp.maximum(jnp.max(sn, axis=-1, keepdims=True), ss)
    en = jnp.exp(sn - mx)
    es = jnp.exp(ss - mx)
    den = es + jnp.sum(en, axis=-1, keepdims=True)
    attn_n = en / den
    attn_s = es / den
    mpre = attn_s * vs + jnp.sum(attn_n[:, :, None] * vn, axis=1)
    m_ref[...] = jnp.dot(mpre, wo_ref[...], preferred_element_type=f32)[None]


_atom3 = lambda n: pl.BlockSpec((1, _BLK, n), lambda b, j: (b * _ABLK + j, 0, 0))
_full = lambda shape: pl.BlockSpec(shape, lambda b, j: tuple(0 for _ in shape))
_wf4 = pl.BlockSpec((1, _BLK, NBH, F), lambda b, j: (b, j, 0, 0))


def _tc_call(interpret=False):
    return pl.pallas_call(
        _tc_body,
        grid=(B, _ABLK),
        in_specs=[
            _atom3(DIN),
            _atom3(DIN),
            pl.BlockSpec((1, 1, DIN), lambda b, j: (b, 0, 0)),
            _atom3(NBH),
            _atom3(NBH),
            pl.BlockSpec((1, _BLK, NBH, NB), lambda b, j: (b, j, 0, 0)),
            pl.BlockSpec((_BLK * NBH, DIN), lambda b, j: (b * _ABLK + j, 0)),
            _full((NB, F)),
            _full((F,)),
            _full((F, F)),
            _full((F,)),
            _full((DIN, F)),
            _full((F, DIN)),
            _full((DIN, F)),
            _full((F, F)),
        ],
        out_specs=[
            _wf4,
            _atom3(F),
        ],
        out_shape=[
            jax.ShapeDtypeStruct((B, A, NBH, F), jnp.float32),
            jax.ShapeDtypeStruct((B * _ABLK, _BLK, F), jnp.float32),
        ],
        interpret=interpret,
    )


def kernel(e, x, t, r_ij, neighbors, pairwise_mask, f_ij, W1, b1, W2, b2, Wq, Wk, Wv, Wo):
    x2d = x.reshape(B * A, DIN)
    idx = neighbors.reshape(ROWS)
    nb = B * _ABLK
    x3 = x.reshape(nb, _BLK, DIN)
    e3 = e.reshape(nb, _BLK, DIN)
    r3 = r_ij.reshape(nb, _BLK, NBH)
    pm3 = pairwise_mask.reshape(nb, _BLK, NBH)

    xn = _sc_gather(x2d, idx)
    wf, m = _tc_call()(
        x3, e3, t.reshape(B, 1, DIN), r3, pm3, f_ij, xn,
        W1, b1, W2, b2, Wq, Wk.T, Wv, Wo)
    return m.reshape(B, A, F), wf
